# Initial kernel scaffold; baseline (speedup 1.0000x reference)
import functools

import jax
import jax.numpy as jnp
from jax import lax
from jax.experimental import pallas as pl
from jax.experimental.pallas import tpu as pltpu
from jax.experimental.pallas import tpu_sc as plsc

F32 = jnp.float32

N = 10000
E = 320000
D_NODE = 128
D_EDGE = 16
D_U = 16
H1 = 128
H2 = 64
D_OUT = 64

NW = 32
CHUNK = 128
NCH = E // CHUNK
NPAD = 10240

EDGE_BLK = 2000
NODE_BLK = 2000


def _pq_body(nf_ref, wp_ref, wq_ref, weu_ref, be1_ref, wnu_ref, bn1_ref,
             g_ref, p_ref, q_ref, cn_ref):
    nf = nf_ref[...]
    ce = jnp.dot(g_ref[...], weu_ref[...], preferred_element_type=F32) + be1_ref[...]
    p_ref[...] = jnp.dot(nf, wp_ref[...], preferred_element_type=F32) + ce
    q_ref[...] = jnp.dot(nf, wq_ref[...], preferred_element_type=F32)
    cn_ref[...] = jnp.dot(g_ref[...], wnu_ref[...], preferred_element_type=F32) + bn1_ref[...]


def _precompute(nf, wp, wq, weu, be1, wnu, bn1, g):
    return pl.pallas_call(
        _pq_body,
        out_shape=[
            jax.ShapeDtypeStruct((N, H1), F32),
            jax.ShapeDtypeStruct((N, H1), F32),
            jax.ShapeDtypeStruct((1, H1), F32),
        ],
    )(nf, wp, wq, weu, be1, wnu, bn1, g)


def _gather_body(p_hbm, q_hbm, src_hbm, dst_hbm, out_hbm,
                 idx_s, idx_d, rp, rq, sem_p, sem_q):
    cid = lax.axis_index("c")
    sid = lax.axis_index("s")
    wid = sid * 2 + cid
    nch_mine = 78 + (wid < NCH - 78 * NW).astype(jnp.int32)

    def body(i, _):
        base = (wid + i * NW) * CHUNK
        pltpu.sync_copy(src_hbm.at[pl.ds(base, CHUNK)], idx_s)
        pltpu.sync_copy(dst_hbm.at[pl.ds(base, CHUNK)], idx_d)
        cp_p = pltpu.async_copy(p_hbm.at[idx_s], rp, sem_p)
        cp_q = pltpu.async_copy(q_hbm.at[idx_d], rq, sem_q)
        cp_p.wait()
        cp_q.wait()

        def add_row(r, _):
            for j in range(H1 // 16):
                sl = pl.ds(j * 16, 16)
                rp[r, sl] = rp[r, sl] + rq[r, sl]
            return 0

        lax.fori_loop(0, CHUNK, add_row, 0, unroll=2)
        pltpu.sync_copy(rp, out_hbm.at[pl.ds(base, CHUNK)])
        return 0

    lax.fori_loop(0, nch_mine, body, 0)


def _sc_gather(p, q, src, dst):
    mesh = plsc.VectorSubcoreMesh(core_axis_name="c", subcore_axis_name="s")
    fn = functools.partial(
        pl.kernel,
        mesh=mesh,
        out_type=jax.ShapeDtypeStruct((E, H1), F32),
        scratch_types=[
            pltpu.VMEM((CHUNK,), jnp.int32),
            pltpu.VMEM((CHUNK,), jnp.int32),
            pltpu.VMEM((CHUNK, H1), F32),
            pltpu.VMEM((CHUNK, H1), F32),
            pltpu.SemaphoreType.DMA,
            pltpu.SemaphoreType.DMA,
        ],
    )(_gather_body)
    return fn(p, q, src, dst)


def _edge_body(g_ref, ef_ref, wa_ref, w2_ref, b2_ref, w3_ref, b3_ref,
               e_ref, ecomb_ref):
    z = g_ref[...] + jnp.dot(ef_ref[...], wa_ref[...], preferred_element_type=F32)
    h1 = jnp.maximum(z, 0.0)
    h2 = jnp.dot(h1, w2_ref[...], preferred_element_type=F32) + b2_ref[...]
    en = jnp.dot(jnp.maximum(h2, 0.0), w3_ref[...], preferred_element_type=F32) + b3_ref[...]
    e_ref[...] = en

    @pl.when(pl.program_id(0) == 0)
    def _init():
        ecomb_ref[...] = jnp.zeros_like(ecomb_ref)

    ecomb_ref[...] += jnp.sum(en, axis=0, keepdims=True)


def _tc_edge(g, ef, wa, w2, b2, w3, b3):
    nblk = E // EDGE_BLK
    return pl.pallas_call(
        _edge_body,
        grid=(nblk,),
        in_specs=[
            pl.BlockSpec((EDGE_BLK, H1), lambda i: (i, 0)),
            pl.BlockSpec((EDGE_BLK, D_EDGE), lambda i: (i, 0)),
            pl.BlockSpec((D_EDGE, H1), lambda i: (0, 0)),
            pl.BlockSpec((H1, H2), lambda i: (0, 0)),
            pl.BlockSpec((1, H2), lambda i: (0, 0)),
            pl.BlockSpec((H2, D_OUT), lambda i: (0, 0)),
            pl.BlockSpec((1, D_OUT), lambda i: (0, 0)),
        ],
        out_specs=[
            pl.BlockSpec((EDGE_BLK, D_OUT), lambda i: (i, 0)),
            pl.BlockSpec((1, D_OUT), lambda i: (0, 0)),
        ],
        out_shape=[
            jax.ShapeDtypeStruct((E, D_OUT), F32),
            jax.ShapeDtypeStruct((1, D_OUT), F32),
        ],
        compiler_params=pltpu.CompilerParams(
            dimension_semantics=("arbitrary",)),
    )(g, ef, wa, w2, b2, w3, b3)


def _scatter_body(e_hbm, dst_hbm, out_hbm, idx_d, rows, zbuf, acc_sh):
    cid = lax.axis_index("c")
    sid = lax.axis_index("s")
    wid = sid * 2 + cid
    zrows = NPAD // 16

    def zrow(r, _):
        for j in range(D_OUT // 16):
            zbuf[r, pl.ds(j * 16, 16)] = jnp.zeros((16,), F32)
        return 0

    lax.fori_loop(0, zrows, zrow, 0, unroll=2)
    pltpu.sync_copy(zbuf, acc_sh.at[pl.ds(sid * zrows, zrows)])
    plsc.subcore_barrier()

    nch_mine = 78 + (wid < NCH - 78 * NW).astype(jnp.int32)

    def body(i, _):
        base = (wid + i * NW) * CHUNK
        pltpu.sync_copy(dst_hbm.at[pl.ds(base, CHUNK)], idx_d)
        pltpu.sync_copy(e_hbm.at[pl.ds(base, CHUNK)], rows)
        pltpu.sync_copy(rows, acc_sh.at[idx_d], add=True)
        return 0

    lax.fori_loop(0, nch_mine, body, 0)
    plsc.subcore_barrier()

    @pl.when(sid < 15)
    def _full():
        pltpu.sync_copy(acc_sh.at[pl.ds(sid * zrows, zrows)],
                        out_hbm.at[cid, pl.ds(sid * zrows, zrows)])

    @pl.when(sid == 15)
    def _tail():
        pltpu.sync_copy(acc_sh.at[pl.ds(15 * zrows, N - 15 * zrows)],
                        out_hbm.at[cid, pl.ds(15 * zrows, N - 15 * zrows)])


def _sc_scatter(e_new, dst):
    mesh = plsc.VectorSubcoreMesh(core_axis_name="c", subcore_axis_name="s")
    fn = functools.partial(
        pl.kernel,
        mesh=mesh,
        out_type=jax.ShapeDtypeStruct((2, N, D_OUT), F32),
        scratch_types=[
            pltpu.VMEM((CHUNK,), jnp.int32),
            pltpu.VMEM((CHUNK, D_OUT), F32),
            pltpu.VMEM((NPAD // 16, D_OUT), F32),
            pltpu.VMEM_SHARED((NPAD, D_OUT), F32),
        ],
    )(_scatter_body)
    return fn(e_new, dst)


def _node_body(nf_ref, m0_ref, m1_ref, w1a_ref, w1b_ref, cn_ref,
               w2_ref, b2_ref, w3_ref, b3_ref,
               ecomb_ref, g_ref, wu1_ref, bu1_ref, wu2_ref, bu2_ref,
               wu3_ref, bu3_ref,
               n_ref, uout_ref, ncomb_acc):
    msgs = m0_ref[...] + m1_ref[...]
    z = (jnp.dot(nf_ref[...], w1a_ref[...], preferred_element_type=F32)
         + jnp.dot(msgs, w1b_ref[...], preferred_element_type=F32)
         + cn_ref[...])
    h1 = jnp.maximum(z, 0.0)
    h2 = jnp.dot(h1, w2_ref[...], preferred_element_type=F32) + b2_ref[...]
    nn = jnp.dot(jnp.maximum(h2, 0.0), w3_ref[...], preferred_element_type=F32) + b3_ref[...]
    n_ref[...] = nn

    @pl.when(pl.program_id(0) == 0)
    def _init():
        ncomb_acc[...] = jnp.zeros_like(ncomb_acc)

    ncomb_acc[...] += jnp.sum(nn, axis=0, keepdims=True)

    @pl.when(pl.program_id(0) == pl.num_programs(0) - 1)
    def _global():
        inp_u = jnp.concatenate(
            [ncomb_acc[...], ecomb_ref[...], g_ref[...]], axis=-1)
        hu = jnp.maximum(jnp.dot(inp_u, wu1_ref[...], preferred_element_type=F32)
                         + bu1_ref[...], 0.0)
        hu = jnp.dot(hu, wu2_ref[...], preferred_element_type=F32) + bu2_ref[...]
        uout_ref[...] = (jnp.dot(jnp.maximum(hu, 0.0), wu3_ref[...],
                                 preferred_element_type=F32) + bu3_ref[...])


def _tc_node(nf, m0, m1, w1a, w1b, cn, w2, b2, w3, b3,
             ecomb, g, wu1, bu1, wu2, bu2, wu3, bu3):
    nblk = N // NODE_BLK
    full = lambda i: (0, 0)
    return pl.pallas_call(
        _node_body,
        grid=(nblk,),
        in_specs=[
            pl.BlockSpec((NODE_BLK, D_NODE), lambda i: (i, 0)),
            pl.BlockSpec((NODE_BLK, D_OUT), lambda i: (i, 0)),
            pl.BlockSpec((NODE_BLK, D_OUT), lambda i: (i, 0)),
            pl.BlockSpec((D_NODE, H1), full),
            pl.BlockSpec((D_OUT, H1), full),
            pl.BlockSpec((1, H1), full),
            pl.BlockSpec((H1, H2), full),
            pl.BlockSpec((1, H2), full),
            pl.BlockSpec((H2, D_OUT), full),
            pl.BlockSpec((1, D_OUT), full),
            pl.BlockSpec((1, D_OUT), full),
            pl.BlockSpec((1, D_U), full),
            pl.BlockSpec((2 * D_OUT + D_U, H1), full),
            pl.BlockSpec((1, H1), full),
            pl.BlockSpec((H1, H2), full),
            pl.BlockSpec((1, H2), full),
            pl.BlockSpec((H2, D_OUT), full),
            pl.BlockSpec((1, D_OUT), full),
        ],
        out_specs=[
            pl.BlockSpec((NODE_BLK, D_OUT), lambda i: (i, 0)),
            pl.BlockSpec((1, D_OUT), full),
        ],
        out_shape=[
            jax.ShapeDtypeStruct((N, D_OUT), F32),
            jax.ShapeDtypeStruct((1, D_OUT), F32),
        ],
        scratch_shapes=[pltpu.VMEM((1, D_OUT), F32)],
        compiler_params=pltpu.CompilerParams(
            dimension_semantics=("arbitrary",)),
    )(nf, m0, m1, w1a, w1b, cn, w2, b2, w3, b3,
      ecomb, g, wu1, bu1, wu2, bu2, wu3, bu3)


def kernel(edge_index, edge_feat, node_feat, g_repr,
           We1, be1, We2, be2, We3, be3,
           Wn1, bn1, Wn2, bn2, Wn3, bn3,
           Wu1, bu1, Wu2, bu2, Wu3, bu3):
    src = edge_index[0].astype(jnp.int32)
    dst = edge_index[1].astype(jnp.int32)

    p, q, cn = _precompute(
        node_feat,
        We1[D_EDGE:D_EDGE + D_NODE],
        We1[D_EDGE + D_NODE:D_EDGE + 2 * D_NODE],
        We1[D_EDGE + 2 * D_NODE:],
        be1.reshape(1, -1),
        Wn1[D_NODE + D_OUT:],
        bn1.reshape(1, -1),
        g_repr,
    )

    g = _sc_gather(p, q, src, dst)

    e_new, e_comb = _tc_edge(g, edge_feat, We1[:D_EDGE],
                             We2, be2.reshape(1, -1), We3, be3.reshape(1, -1))

    msgs2 = _sc_scatter(e_new, dst)

    n_new, u_out = _tc_node(
        node_feat, msgs2[0], msgs2[1],
        Wn1[:D_NODE], Wn1[D_NODE:D_NODE + D_OUT], cn,
        Wn2, bn2.reshape(1, -1), Wn3, bn3.reshape(1, -1),
        e_comb, g_repr,
        Wu1, bu1.reshape(1, -1), Wu2, bu2.reshape(1, -1),
        Wu3, bu3.reshape(1, -1),
    )
    return (e_new, n_new, u_out)


# speedup vs baseline: 1.9121x; 1.9121x over previous
import functools

import jax
import jax.numpy as jnp
from jax import lax
from jax.experimental import pallas as pl
from jax.experimental.pallas import tpu as pltpu
from jax.experimental.pallas import tpu_sc as plsc

F32 = jnp.float32

N = 10000
E = 320000
D_NODE = 128
D_EDGE = 16
D_U = 16
H1 = 128
H2 = 64
D_OUT = 64

NW = 32
CG = 64
NCHG = E // CG
NPAD = 10240

EDGE_BLK = 2000
NODE_BLK = 2000


def _pq_body(nf_ref, wp_ref, wq_ref, weu_ref, be1_ref, wnu_ref, bn1_ref,
             g_ref, p_ref, q_ref, cn_ref):
    nf = nf_ref[...]
    ce = jnp.dot(g_ref[...], weu_ref[...], preferred_element_type=F32) + be1_ref[...]
    p_ref[...] = jnp.dot(nf, wp_ref[...], preferred_element_type=F32) + ce
    q_ref[...] = jnp.dot(nf, wq_ref[...], preferred_element_type=F32)
    cn_ref[...] = jnp.dot(g_ref[...], wnu_ref[...], preferred_element_type=F32) + bn1_ref[...]


def _precompute(nf, wp, wq, weu, be1, wnu, bn1, g):
    return pl.pallas_call(
        _pq_body,
        out_shape=[
            jax.ShapeDtypeStruct((N, H1), F32),
            jax.ShapeDtypeStruct((N, H1), F32),
            jax.ShapeDtypeStruct((1, H1), F32),
        ],
    )(nf, wp, wq, weu, be1, wnu, bn1, g)


def _gather_body(p_hbm, q_hbm, src_hbm, dst_hbm, out_hbm,
                 idx_s, idx_d, rp, rq, sem_p, sem_q):
    cid = lax.axis_index("c")
    sid = lax.axis_index("s")
    wid = sid * 2 + cid
    nch_mine = (NCHG // NW) + (wid < NCHG - (NCHG // NW) * NW).astype(jnp.int32)

    def body(i, _):
        base = (wid + i * NW) * CG
        pltpu.sync_copy(src_hbm.at[pl.ds(base, CG)], idx_s)
        pltpu.sync_copy(dst_hbm.at[pl.ds(base, CG)], idx_d)
        cp_p = pltpu.async_copy(p_hbm.at[idx_s], rp, sem_p)
        cp_q = pltpu.async_copy(q_hbm.at[idx_d], rq, sem_q)
        cp_p.wait()
        cp_q.wait()

        def add_row(r, _):
            for j in range(H1 // 16):
                sl = pl.ds(j * 16, 16)
                rp[r, sl] = rp[r, sl] + rq[r, sl]
            return 0

        lax.fori_loop(0, CG, add_row, 0, unroll=2)
        pltpu.sync_copy(rp, out_hbm.at[pl.ds(base, CG)])
        return 0

    lax.fori_loop(0, nch_mine, body, 0)


def _sc_gather(p, q, src, dst):
    mesh = plsc.VectorSubcoreMesh(core_axis_name="c", subcore_axis_name="s")
    fn = functools.partial(
        pl.kernel,
        mesh=mesh,
        out_type=jax.ShapeDtypeStruct((E, H1), F32),
        scratch_types=[
            pltpu.VMEM((CG,), jnp.int32),
            pltpu.VMEM((CG,), jnp.int32),
            pltpu.VMEM((CG, H1), F32),
            pltpu.VMEM((CG, H1), F32),
            pltpu.SemaphoreType.DMA,
            pltpu.SemaphoreType.DMA,
        ],
    )(_gather_body)
    return fn(p, q, src, dst)


def _edge_body(g_ref, ef_ref, wa_ref, w2_ref, b2_ref, w3_ref, b3_ref,
               e_ref, epad_ref, ecomb_ref):
    z = g_ref[...] + jnp.dot(ef_ref[...], wa_ref[...], preferred_element_type=F32)
    h1 = jnp.maximum(z, 0.0)
    h2 = jnp.dot(h1, w2_ref[...], preferred_element_type=F32) + b2_ref[...]
    en = jnp.dot(jnp.maximum(h2, 0.0), w3_ref[...], preferred_element_type=F32) + b3_ref[...]
    e_ref[...] = en
    epad_ref[...] = jnp.concatenate([en, jnp.zeros_like(en)], axis=-1)

    @pl.when(pl.program_id(0) == 0)
    def _init():
        ecomb_ref[...] = jnp.zeros_like(ecomb_ref)

    ecomb_ref[...] += jnp.sum(en, axis=0, keepdims=True)


def _tc_edge(g, ef, wa, w2, b2, w3, b3):
    nblk = E // EDGE_BLK
    return pl.pallas_call(
        _edge_body,
        grid=(nblk,),
        in_specs=[
            pl.BlockSpec((EDGE_BLK, H1), lambda i: (i, 0)),
            pl.BlockSpec((EDGE_BLK, D_EDGE), lambda i: (i, 0)),
            pl.BlockSpec((D_EDGE, H1), lambda i: (0, 0)),
            pl.BlockSpec((H1, H2), lambda i: (0, 0)),
            pl.BlockSpec((1, H2), lambda i: (0, 0)),
            pl.BlockSpec((H2, D_OUT), lambda i: (0, 0)),
            pl.BlockSpec((1, D_OUT), lambda i: (0, 0)),
        ],
        out_specs=[
            pl.BlockSpec((EDGE_BLK, D_OUT), lambda i: (i, 0)),
            pl.BlockSpec((EDGE_BLK, 2 * D_OUT), lambda i: (i, 0)),
            pl.BlockSpec((1, D_OUT), lambda i: (0, 0)),
        ],
        out_shape=[
            jax.ShapeDtypeStruct((E, D_OUT), F32),
            jax.ShapeDtypeStruct((E, 2 * D_OUT), F32),
            jax.ShapeDtypeStruct((1, D_OUT), F32),
        ],
        compiler_params=pltpu.CompilerParams(
            dimension_semantics=("arbitrary",)),
    )(g, ef, wa, w2, b2, w3, b3)


CS = 64
NCHS = E // CS


def _scatter_body(e_hbm, dst_hbm, out_hbm, idx_d, rows, acc_sh):
    cid = lax.axis_index("c")
    sid = lax.axis_index("s")
    wid = sid * 2 + cid
    zrows = NPAD // 16

    def zrow(r, _):
        for j in range(8):
            rows[r, pl.ds(j * 16, 16)] = jnp.zeros((16,), F32)
        return 0

    lax.fori_loop(0, CS, zrow, 0, unroll=2)

    def zcopy(k, _):
        pltpu.sync_copy(rows, acc_sh.at[pl.ds(sid * zrows + k * CS, CS)])
        return 0

    lax.fori_loop(0, zrows // CS, zcopy, 0)
    plsc.subcore_barrier()

    nch_mine = (NCHS // NW) + (wid < NCHS - (NCHS // NW) * NW).astype(jnp.int32)

    def body(i, _):
        base = (wid + i * NW) * CS
        pltpu.sync_copy(dst_hbm.at[pl.ds(base, CS)], idx_d)
        pltpu.sync_copy(e_hbm.at[pl.ds(base, CS)], rows)
        pltpu.sync_copy(rows, acc_sh.at[idx_d], add=True)
        return 0

    lax.fori_loop(0, nch_mine, body, 0)
    plsc.subcore_barrier()

    @pl.when(sid < 15)
    def _full():
        pltpu.sync_copy(acc_sh.at[pl.ds(sid * zrows, zrows)],
                        out_hbm.at[cid, pl.ds(sid * zrows, zrows)])

    @pl.when(sid == 15)
    def _tail():
        pltpu.sync_copy(acc_sh.at[pl.ds(15 * zrows, N - 15 * zrows)],
                        out_hbm.at[cid, pl.ds(15 * zrows, N - 15 * zrows)])


def _sc_scatter(e_pad, dst):
    mesh = plsc.VectorSubcoreMesh(core_axis_name="c", subcore_axis_name="s")
    fn = functools.partial(
        pl.kernel,
        mesh=mesh,
        out_type=jax.ShapeDtypeStruct((2, N, 2 * D_OUT), F32),
        scratch_types=[
            pltpu.VMEM((CS,), jnp.int32),
            pltpu.VMEM((CS, 2 * D_OUT), F32),
            pltpu.VMEM_SHARED((NPAD, 2 * D_OUT), F32),
        ],
    )(_scatter_body)
    return fn(e_pad, dst)


def _node_body(nf_ref, m0_ref, m1_ref, w1a_ref, w1b_ref, cn_ref,
               w2_ref, b2_ref, w3_ref, b3_ref,
               ecomb_ref, g_ref, wu1_ref, bu1_ref, wu2_ref, bu2_ref,
               wu3_ref, bu3_ref,
               n_ref, uout_ref, ncomb_acc):
    msgs = (m0_ref[...] + m1_ref[...])[:, :D_OUT]
    z = (jnp.dot(nf_ref[...], w1a_ref[...], preferred_element_type=F32)
         + jnp.dot(msgs, w1b_ref[...], preferred_element_type=F32)
         + cn_ref[...])
    h1 = jnp.maximum(z, 0.0)
    h2 = jnp.dot(h1, w2_ref[...], preferred_element_type=F32) + b2_ref[...]
    nn = jnp.dot(jnp.maximum(h2, 0.0), w3_ref[...], preferred_element_type=F32) + b3_ref[...]
    n_ref[...] = nn

    @pl.when(pl.program_id(0) == 0)
    def _init():
        ncomb_acc[...] = jnp.zeros_like(ncomb_acc)

    ncomb_acc[...] += jnp.sum(nn, axis=0, keepdims=True)

    @pl.when(pl.program_id(0) == pl.num_programs(0) - 1)
    def _global():
        inp_u = jnp.concatenate(
            [ncomb_acc[...], ecomb_ref[...], g_ref[...]], axis=-1)
        hu = jnp.maximum(jnp.dot(inp_u, wu1_ref[...], preferred_element_type=F32)
                         + bu1_ref[...], 0.0)
        hu = jnp.dot(hu, wu2_ref[...], preferred_element_type=F32) + bu2_ref[...]
        uout_ref[...] = (jnp.dot(jnp.maximum(hu, 0.0), wu3_ref[...],
                                 preferred_element_type=F32) + bu3_ref[...])


def _tc_node(nf, m0, m1, w1a, w1b, cn, w2, b2, w3, b3,
             ecomb, g, wu1, bu1, wu2, bu2, wu3, bu3):
    nblk = N // NODE_BLK
    full = lambda i: (0, 0)
    return pl.pallas_call(
        _node_body,
        grid=(nblk,),
        in_specs=[
            pl.BlockSpec((NODE_BLK, D_NODE), lambda i: (i, 0)),
            pl.BlockSpec((NODE_BLK, 2 * D_OUT), lambda i: (i, 0)),
            pl.BlockSpec((NODE_BLK, 2 * D_OUT), lambda i: (i, 0)),
            pl.BlockSpec((D_NODE, H1), full),
            pl.BlockSpec((D_OUT, H1), full),
            pl.BlockSpec((1, H1), full),
            pl.BlockSpec((H1, H2), full),
            pl.BlockSpec((1, H2), full),
            pl.BlockSpec((H2, D_OUT), full),
            pl.BlockSpec((1, D_OUT), full),
            pl.BlockSpec((1, D_OUT), full),
            pl.BlockSpec((1, D_U), full),
            pl.BlockSpec((2 * D_OUT + D_U, H1), full),
            pl.BlockSpec((1, H1), full),
            pl.BlockSpec((H1, H2), full),
            pl.BlockSpec((1, H2), full),
            pl.BlockSpec((H2, D_OUT), full),
            pl.BlockSpec((1, D_OUT), full),
        ],
        out_specs=[
            pl.BlockSpec((NODE_BLK, D_OUT), lambda i: (i, 0)),
            pl.BlockSpec((1, D_OUT), full),
        ],
        out_shape=[
            jax.ShapeDtypeStruct((N, D_OUT), F32),
            jax.ShapeDtypeStruct((1, D_OUT), F32),
        ],
        scratch_shapes=[pltpu.VMEM((1, D_OUT), F32)],
        compiler_params=pltpu.CompilerParams(
            dimension_semantics=("arbitrary",)),
    )(nf, m0, m1, w1a, w1b, cn, w2, b2, w3, b3,
      ecomb, g, wu1, bu1, wu2, bu2, wu3, bu3)


def kernel(edge_index, edge_feat, node_feat, g_repr,
           We1, be1, We2, be2, We3, be3,
           Wn1, bn1, Wn2, bn2, Wn3, bn3,
           Wu1, bu1, Wu2, bu2, Wu3, bu3):
    src = edge_index[0].astype(jnp.int32)
    dst = edge_index[1].astype(jnp.int32)

    p, q, cn = _precompute(
        node_feat,
        We1[D_EDGE:D_EDGE + D_NODE],
        We1[D_EDGE + D_NODE:D_EDGE + 2 * D_NODE],
        We1[D_EDGE + 2 * D_NODE:],
        be1.reshape(1, -1),
        Wn1[D_NODE + D_OUT:],
        bn1.reshape(1, -1),
        g_repr,
    )

    g = _sc_gather(p, q, src, dst)

    e_new, e_pad, e_comb = _tc_edge(g, edge_feat, We1[:D_EDGE],
                                    We2, be2.reshape(1, -1),
                                    We3, be3.reshape(1, -1))

    msgs2 = _sc_scatter(e_pad, dst)

    n_new, u_out = _tc_node(
        node_feat, msgs2[0], msgs2[1],
        Wn1[:D_NODE], Wn1[D_NODE:D_NODE + D_OUT], cn,
        Wn2, bn2.reshape(1, -1), Wn3, bn3.reshape(1, -1),
        e_comb, g_repr,
        Wu1, bu1.reshape(1, -1), Wu2, bu2.reshape(1, -1),
        Wu3, bu3.reshape(1, -1),
    )
    return (e_new, n_new, u_out)


# speedup vs baseline: 2.2921x; 1.1988x over previous
import functools

import jax
import jax.numpy as jnp
from jax import lax
from jax.experimental import pallas as pl
from jax.experimental.pallas import tpu as pltpu
from jax.experimental.pallas import tpu_sc as plsc

F32 = jnp.float32

N = 10000
E = 320000
D_NODE = 128
D_EDGE = 16
D_U = 16
H1 = 128
H2 = 64
D_OUT = 64

NW = 32
CG = 40
NCHG = E // CG
NPAD = 10240

EDGE_BLK = 2000
NODE_BLK = 2000


def _pq_body(nf_ref, wp_ref, wq_ref, weu_ref, be1_ref, wnu_ref, bn1_ref,
             g_ref, p_ref, q_ref, cn_ref):
    nf = nf_ref[...]
    ce = jnp.dot(g_ref[...], weu_ref[...], preferred_element_type=F32) + be1_ref[...]
    p_ref[...] = jnp.dot(nf, wp_ref[...], preferred_element_type=F32) + ce
    q_ref[...] = jnp.dot(nf, wq_ref[...], preferred_element_type=F32)
    cn_ref[...] = jnp.dot(g_ref[...], wnu_ref[...], preferred_element_type=F32) + bn1_ref[...]


def _precompute(nf, wp, wq, weu, be1, wnu, bn1, g):
    return pl.pallas_call(
        _pq_body,
        out_shape=[
            jax.ShapeDtypeStruct((N, H1), F32),
            jax.ShapeDtypeStruct((N, H1), F32),
            jax.ShapeDtypeStruct((1, H1), F32),
        ],
    )(nf, wp, wq, weu, be1, wnu, bn1, g)


def _gather_body(p_hbm, q_hbm, src_hbm, dst_hbm, out_hbm,
                 is0, is1, id0, id1, rp0, rq0, rp1, rq1,
                 si0, si1, sp0, sp1, sq0, sq1, sw0, sw1):
    cid = lax.axis_index("c")
    sid = lax.axis_index("s")
    wid = sid * 2 + cid
    IS, ID = (is0, is1), (id0, id1)
    RP, RQ = (rp0, rp1), (rq0, rq1)
    SI, SP, SQ, SW = (si0, si1), (sp0, sp1), (sq0, sq1), (sw0, sw1)
    PT = NCHG // NW

    def ebase(c):
        return (wid + c * NW) * CG

    def issue_idx(c, s):
        pltpu.make_async_copy(src_hbm.at[pl.ds(ebase(c), CG)], IS[s], SI[s]).start()
        pltpu.make_async_copy(dst_hbm.at[pl.ds(ebase(c), CG)], ID[s], SI[s]).start()

    def wait_idx(c, s):
        pltpu.make_async_copy(src_hbm.at[pl.ds(ebase(c), CG)], IS[s], SI[s]).wait()
        pltpu.make_async_copy(dst_hbm.at[pl.ds(ebase(c), CG)], ID[s], SI[s]).wait()

    def issue_gather(s):
        pltpu.make_async_copy(p_hbm.at[IS[s]], RP[s], SP[s]).start()
        pltpu.make_async_copy(q_hbm.at[ID[s]], RQ[s], SQ[s]).start()

    def wait_gather(s):
        pltpu.make_async_copy(p_hbm.at[IS[s]], RP[s], SP[s]).wait()
        pltpu.make_async_copy(q_hbm.at[ID[s]], RQ[s], SQ[s]).wait()

    def add(s):
        rp, rq = RP[s], RQ[s]

        def add_row(r, _):
            for j in range(H1 // 16):
                sl = pl.ds(j * 16, 16)
                rp[r, sl] = rp[r, sl] + rq[r, sl]
            return 0

        lax.fori_loop(0, CG, add_row, 0, unroll=2)

    def issue_write(c, s):
        pltpu.make_async_copy(RP[s], out_hbm.at[pl.ds(ebase(c), CG)], SW[s]).start()

    def wait_write(c, s):
        pltpu.make_async_copy(RP[s], out_hbm.at[pl.ds(ebase(c), CG)], SW[s]).wait()

    issue_idx(0, 0)
    issue_idx(1, 1)
    wait_idx(0, 0)
    issue_gather(0)

    def body(t, _):
        a = 2 * t
        b = a + 1
        wait_idx(b, 1)

        @pl.when(t > 0)
        def _wb1():
            wait_write(b - 2, 1)

        issue_gather(1)
        wait_gather(0)

        @pl.when(a + 2 < PT)
        def _ia():
            issue_idx(a + 2, 0)

        add(0)
        issue_write(a, 0)
        wait_gather(1)

        @pl.when(b + 2 < PT)
        def _ib():
            issue_idx(b + 2, 1)

        add(1)
        issue_write(b, 1)

        @pl.when(a + 2 < PT)
        def _ga():
            wait_idx(a + 2, 0)
            wait_write(a, 0)
            issue_gather(0)

        return 0

    lax.fori_loop(0, PT // 2, body, 0)
    wait_write(PT - 2, 0)
    wait_write(PT - 1, 1)


def _sc_gather(p, q, src, dst):
    mesh = plsc.VectorSubcoreMesh(core_axis_name="c", subcore_axis_name="s")
    fn = functools.partial(
        pl.kernel,
        mesh=mesh,
        out_type=jax.ShapeDtypeStruct((E, H1), F32),
        scratch_types=[
            pltpu.VMEM((CG,), jnp.int32),
            pltpu.VMEM((CG,), jnp.int32),
            pltpu.VMEM((CG,), jnp.int32),
            pltpu.VMEM((CG,), jnp.int32),
            pltpu.VMEM((CG, H1), F32),
            pltpu.VMEM((CG, H1), F32),
            pltpu.VMEM((CG, H1), F32),
            pltpu.VMEM((CG, H1), F32),
            pltpu.SemaphoreType.DMA,
            pltpu.SemaphoreType.DMA,
            pltpu.SemaphoreType.DMA,
            pltpu.SemaphoreType.DMA,
            pltpu.SemaphoreType.DMA,
            pltpu.SemaphoreType.DMA,
            pltpu.SemaphoreType.DMA,
            pltpu.SemaphoreType.DMA,
        ],
    )(_gather_body)
    return fn(p, q, src, dst)


def _edge_body(g_ref, ef_ref, wa_ref, w2_ref, b2_ref, w3_ref, b3_ref,
               e_ref, epad_ref, ecomb_ref):
    z = g_ref[...] + jnp.dot(ef_ref[...], wa_ref[...], preferred_element_type=F32)
    h1 = jnp.maximum(z, 0.0)
    h2 = jnp.dot(h1, w2_ref[...], preferred_element_type=F32) + b2_ref[...]
    en = jnp.dot(jnp.maximum(h2, 0.0), w3_ref[...], preferred_element_type=F32) + b3_ref[...]
    e_ref[...] = en
    epad_ref[...] = jnp.concatenate([en, jnp.zeros_like(en)], axis=-1)

    @pl.when(pl.program_id(0) == 0)
    def _init():
        ecomb_ref[...] = jnp.zeros_like(ecomb_ref)

    ecomb_ref[...] += jnp.sum(en, axis=0, keepdims=True)


def _tc_edge(g, ef, wa, w2, b2, w3, b3):
    nblk = E // EDGE_BLK
    return pl.pallas_call(
        _edge_body,
        grid=(nblk,),
        in_specs=[
            pl.BlockSpec((EDGE_BLK, H1), lambda i: (i, 0)),
            pl.BlockSpec((EDGE_BLK, D_EDGE), lambda i: (i, 0)),
            pl.BlockSpec((D_EDGE, H1), lambda i: (0, 0)),
            pl.BlockSpec((H1, H2), lambda i: (0, 0)),
            pl.BlockSpec((1, H2), lambda i: (0, 0)),
            pl.BlockSpec((H2, D_OUT), lambda i: (0, 0)),
            pl.BlockSpec((1, D_OUT), lambda i: (0, 0)),
        ],
        out_specs=[
            pl.BlockSpec((EDGE_BLK, D_OUT), lambda i: (i, 0)),
            pl.BlockSpec((EDGE_BLK, 2 * D_OUT), lambda i: (i, 0)),
            pl.BlockSpec((1, D_OUT), lambda i: (0, 0)),
        ],
        out_shape=[
            jax.ShapeDtypeStruct((E, D_OUT), F32),
            jax.ShapeDtypeStruct((E, 2 * D_OUT), F32),
            jax.ShapeDtypeStruct((1, D_OUT), F32),
        ],
        compiler_params=pltpu.CompilerParams(
            dimension_semantics=("arbitrary",)),
    )(g, ef, wa, w2, b2, w3, b3)


CS = 64
NCHS = E // CS


def _scatter_body(e_hbm, dst_hbm, out_hbm, idx_d, rows, acc_sh):
    cid = lax.axis_index("c")
    sid = lax.axis_index("s")
    wid = sid * 2 + cid
    zrows = NPAD // 16

    def zrow(r, _):
        for j in range(8):
            rows[r, pl.ds(j * 16, 16)] = jnp.zeros((16,), F32)
        return 0

    lax.fori_loop(0, CS, zrow, 0, unroll=2)

    def zcopy(k, _):
        pltpu.sync_copy(rows, acc_sh.at[pl.ds(sid * zrows + k * CS, CS)])
        return 0

    lax.fori_loop(0, zrows // CS, zcopy, 0)
    plsc.subcore_barrier()

    nch_mine = (NCHS // NW) + (wid < NCHS - (NCHS // NW) * NW).astype(jnp.int32)

    def body(i, _):
        base = (wid + i * NW) * CS
        pltpu.sync_copy(dst_hbm.at[pl.ds(base, CS)], idx_d)
        pltpu.sync_copy(e_hbm.at[pl.ds(base, CS)], rows)
        pltpu.sync_copy(rows, acc_sh.at[idx_d], add=True)
        return 0

    lax.fori_loop(0, nch_mine, body, 0)
    plsc.subcore_barrier()

    @pl.when(sid < 15)
    def _full():
        pltpu.sync_copy(acc_sh.at[pl.ds(sid * zrows, zrows)],
                        out_hbm.at[cid, pl.ds(sid * zrows, zrows)])

    @pl.when(sid == 15)
    def _tail():
        pltpu.sync_copy(acc_sh.at[pl.ds(15 * zrows, N - 15 * zrows)],
                        out_hbm.at[cid, pl.ds(15 * zrows, N - 15 * zrows)])


def _sc_scatter(e_pad, dst):
    mesh = plsc.VectorSubcoreMesh(core_axis_name="c", subcore_axis_name="s")
    fn = functools.partial(
        pl.kernel,
        mesh=mesh,
        out_type=jax.ShapeDtypeStruct((2, N, 2 * D_OUT), F32),
        scratch_types=[
            pltpu.VMEM((CS,), jnp.int32),
            pltpu.VMEM((CS, 2 * D_OUT), F32),
            pltpu.VMEM_SHARED((NPAD, 2 * D_OUT), F32),
        ],
    )(_scatter_body)
    return fn(e_pad, dst)


def _node_body(nf_ref, m0_ref, m1_ref, w1a_ref, w1b_ref, cn_ref,
               w2_ref, b2_ref, w3_ref, b3_ref,
               ecomb_ref, g_ref, wu1_ref, bu1_ref, wu2_ref, bu2_ref,
               wu3_ref, bu3_ref,
               n_ref, uout_ref, ncomb_acc):
    msgs = (m0_ref[...] + m1_ref[...])[:, :D_OUT]
    z = (jnp.dot(nf_ref[...], w1a_ref[...], preferred_element_type=F32)
         + jnp.dot(msgs, w1b_ref[...], preferred_element_type=F32)
         + cn_ref[...])
    h1 = jnp.maximum(z, 0.0)
    h2 = jnp.dot(h1, w2_ref[...], preferred_element_type=F32) + b2_ref[...]
    nn = jnp.dot(jnp.maximum(h2, 0.0), w3_ref[...], preferred_element_type=F32) + b3_ref[...]
    n_ref[...] = nn

    @pl.when(pl.program_id(0) == 0)
    def _init():
        ncomb_acc[...] = jnp.zeros_like(ncomb_acc)

    ncomb_acc[...] += jnp.sum(nn, axis=0, keepdims=True)

    @pl.when(pl.program_id(0) == pl.num_programs(0) - 1)
    def _global():
        inp_u = jnp.concatenate(
            [ncomb_acc[...], ecomb_ref[...], g_ref[...]], axis=-1)
        hu = jnp.maximum(jnp.dot(inp_u, wu1_ref[...], preferred_element_type=F32)
                         + bu1_ref[...], 0.0)
        hu = jnp.dot(hu, wu2_ref[...], preferred_element_type=F32) + bu2_ref[...]
        uout_ref[...] = (jnp.dot(jnp.maximum(hu, 0.0), wu3_ref[...],
                                 preferred_element_type=F32) + bu3_ref[...])


def _tc_node(nf, m0, m1, w1a, w1b, cn, w2, b2, w3, b3,
             ecomb, g, wu1, bu1, wu2, bu2, wu3, bu3):
    nblk = N // NODE_BLK
    full = lambda i: (0, 0)
    return pl.pallas_call(
        _node_body,
        grid=(nblk,),
        in_specs=[
            pl.BlockSpec((NODE_BLK, D_NODE), lambda i: (i, 0)),
            pl.BlockSpec((NODE_BLK, 2 * D_OUT), lambda i: (i, 0)),
            pl.BlockSpec((NODE_BLK, 2 * D_OUT), lambda i: (i, 0)),
            pl.BlockSpec((D_NODE, H1), full),
            pl.BlockSpec((D_OUT, H1), full),
            pl.BlockSpec((1, H1), full),
            pl.BlockSpec((H1, H2), full),
            pl.BlockSpec((1, H2), full),
            pl.BlockSpec((H2, D_OUT), full),
            pl.BlockSpec((1, D_OUT), full),
            pl.BlockSpec((1, D_OUT), full),
            pl.BlockSpec((1, D_U), full),
            pl.BlockSpec((2 * D_OUT + D_U, H1), full),
            pl.BlockSpec((1, H1), full),
            pl.BlockSpec((H1, H2), full),
            pl.BlockSpec((1, H2), full),
            pl.BlockSpec((H2, D_OUT), full),
            pl.BlockSpec((1, D_OUT), full),
        ],
        out_specs=[
            pl.BlockSpec((NODE_BLK, D_OUT), lambda i: (i, 0)),
            pl.BlockSpec((1, D_OUT), full),
        ],
        out_shape=[
            jax.ShapeDtypeStruct((N, D_OUT), F32),
            jax.ShapeDtypeStruct((1, D_OUT), F32),
        ],
        scratch_shapes=[pltpu.VMEM((1, D_OUT), F32)],
        compiler_params=pltpu.CompilerParams(
            dimension_semantics=("arbitrary",)),
    )(nf, m0, m1, w1a, w1b, cn, w2, b2, w3, b3,
      ecomb, g, wu1, bu1, wu2, bu2, wu3, bu3)


def kernel(edge_index, edge_feat, node_feat, g_repr,
           We1, be1, We2, be2, We3, be3,
           Wn1, bn1, Wn2, bn2, Wn3, bn3,
           Wu1, bu1, Wu2, bu2, Wu3, bu3):
    src = edge_index[0].astype(jnp.int32)
    dst = edge_index[1].astype(jnp.int32)

    p, q, cn = _precompute(
        node_feat,
        We1[D_EDGE:D_EDGE + D_NODE],
        We1[D_EDGE + D_NODE:D_EDGE + 2 * D_NODE],
        We1[D_EDGE + 2 * D_NODE:],
        be1.reshape(1, -1),
        Wn1[D_NODE + D_OUT:],
        bn1.reshape(1, -1),
        g_repr,
    )

    g = _sc_gather(p, q, src, dst)

    e_new, e_pad, e_comb = _tc_edge(g, edge_feat, We1[:D_EDGE],
                                    We2, be2.reshape(1, -1),
                                    We3, be3.reshape(1, -1))

    msgs2 = _sc_scatter(e_pad, dst)

    n_new, u_out = _tc_node(
        node_feat, msgs2[0], msgs2[1],
        Wn1[:D_NODE], Wn1[D_NODE:D_NODE + D_OUT], cn,
        Wn2, bn2.reshape(1, -1), Wn3, bn3.reshape(1, -1),
        e_comb, g_repr,
        Wu1, bu1.reshape(1, -1), Wu2, bu2.reshape(1, -1),
        Wu3, bu3.reshape(1, -1),
    )
    return (e_new, n_new, u_out)


# speedup vs baseline: 2.5237x; 1.1010x over previous
import functools

import jax
import jax.numpy as jnp
from jax import lax
from jax.experimental import pallas as pl
from jax.experimental.pallas import tpu as pltpu
from jax.experimental.pallas import tpu_sc as plsc

F32 = jnp.float32

N = 10000
E = 320000
D_NODE = 128
D_EDGE = 16
D_U = 16
H1 = 128
H2 = 64
D_OUT = 64

NW = 32
CG = 40
NCHG = E // CG
NPAD = 10240

EDGE_BLK = 2000
NODE_BLK = 2000


def _pq_body(nf_ref, wp_ref, wq_ref, weu_ref, be1_ref, wnu_ref, bn1_ref,
             g_ref, p_ref, q_ref, cn_ref):
    nf = nf_ref[...]
    ce = jnp.dot(g_ref[...], weu_ref[...], preferred_element_type=F32) + be1_ref[...]
    p_ref[...] = jnp.dot(nf, wp_ref[...], preferred_element_type=F32) + ce
    q_ref[...] = jnp.dot(nf, wq_ref[...], preferred_element_type=F32)
    cn_ref[...] = jnp.dot(g_ref[...], wnu_ref[...], preferred_element_type=F32) + bn1_ref[...]


def _precompute(nf, wp, wq, weu, be1, wnu, bn1, g):
    return pl.pallas_call(
        _pq_body,
        out_shape=[
            jax.ShapeDtypeStruct((N, H1), F32),
            jax.ShapeDtypeStruct((N, H1), F32),
            jax.ShapeDtypeStruct((1, H1), F32),
        ],
    )(nf, wp, wq, weu, be1, wnu, bn1, g)


def _gather_body(p_hbm, q_hbm, src_hbm, dst_hbm, out_hbm,
                 is0, is1, id0, id1, rp0, rq0, rp1, rq1,
                 si0, si1, sp0, sp1, sq0, sq1, sw0, sw1):
    cid = lax.axis_index("c")
    sid = lax.axis_index("s")
    wid = sid * 2 + cid
    IS, ID = (is0, is1), (id0, id1)
    RP, RQ = (rp0, rp1), (rq0, rq1)
    SI, SP, SQ, SW = (si0, si1), (sp0, sp1), (sq0, sq1), (sw0, sw1)
    PT = NCHG // NW

    def ebase(c):
        return (wid + c * NW) * CG

    def issue_idx(c, s):
        pltpu.make_async_copy(src_hbm.at[pl.ds(ebase(c), CG)], IS[s], SI[s]).start()
        pltpu.make_async_copy(dst_hbm.at[pl.ds(ebase(c), CG)], ID[s], SI[s]).start()

    def wait_idx(c, s):
        pltpu.make_async_copy(src_hbm.at[pl.ds(ebase(c), CG)], IS[s], SI[s]).wait()
        pltpu.make_async_copy(dst_hbm.at[pl.ds(ebase(c), CG)], ID[s], SI[s]).wait()

    def issue_gather(s):
        pltpu.make_async_copy(p_hbm.at[IS[s]], RP[s], SP[s]).start()
        pltpu.make_async_copy(q_hbm.at[ID[s]], RQ[s], SQ[s]).start()

    def wait_gather(s):
        pltpu.make_async_copy(p_hbm.at[IS[s]], RP[s], SP[s]).wait()
        pltpu.make_async_copy(q_hbm.at[ID[s]], RQ[s], SQ[s]).wait()

    def add(s):
        rp, rq = RP[s], RQ[s]

        def add_row(r, _):
            for j in range(H1 // 16):
                sl = pl.ds(j * 16, 16)
                rp[r, sl] = rp[r, sl] + rq[r, sl]
            return 0

        lax.fori_loop(0, CG, add_row, 0, unroll=2)

    def issue_write(c, s):
        pltpu.make_async_copy(RP[s], out_hbm.at[pl.ds(ebase(c), CG)], SW[s]).start()

    def wait_write(c, s):
        pltpu.make_async_copy(RP[s], out_hbm.at[pl.ds(ebase(c), CG)], SW[s]).wait()

    issue_idx(0, 0)
    issue_idx(1, 1)
    wait_idx(0, 0)
    issue_gather(0)

    def body(t, _):
        a = 2 * t
        b = a + 1
        wait_idx(b, 1)

        @pl.when(t > 0)
        def _wb1():
            wait_write(b - 2, 1)

        issue_gather(1)
        wait_gather(0)

        @pl.when(a + 2 < PT)
        def _ia():
            issue_idx(a + 2, 0)

        add(0)
        issue_write(a, 0)
        wait_gather(1)

        @pl.when(b + 2 < PT)
        def _ib():
            issue_idx(b + 2, 1)

        add(1)
        issue_write(b, 1)

        @pl.when(a + 2 < PT)
        def _ga():
            wait_idx(a + 2, 0)
            wait_write(a, 0)
            issue_gather(0)

        return 0

    lax.fori_loop(0, PT // 2, body, 0)
    wait_write(PT - 2, 0)
    wait_write(PT - 1, 1)


def _sc_gather(p, q, src, dst):
    mesh = plsc.VectorSubcoreMesh(core_axis_name="c", subcore_axis_name="s")
    fn = functools.partial(
        pl.kernel,
        mesh=mesh,
        out_type=jax.ShapeDtypeStruct((E, H1), F32),
        scratch_types=[
            pltpu.VMEM((CG,), jnp.int32),
            pltpu.VMEM((CG,), jnp.int32),
            pltpu.VMEM((CG,), jnp.int32),
            pltpu.VMEM((CG,), jnp.int32),
            pltpu.VMEM((CG, H1), F32),
            pltpu.VMEM((CG, H1), F32),
            pltpu.VMEM((CG, H1), F32),
            pltpu.VMEM((CG, H1), F32),
            pltpu.SemaphoreType.DMA,
            pltpu.SemaphoreType.DMA,
            pltpu.SemaphoreType.DMA,
            pltpu.SemaphoreType.DMA,
            pltpu.SemaphoreType.DMA,
            pltpu.SemaphoreType.DMA,
            pltpu.SemaphoreType.DMA,
            pltpu.SemaphoreType.DMA,
        ],
    )(_gather_body)
    return fn(p, q, src, dst)


def _edge_body(g_ref, ef_ref, wa_ref, w2_ref, b2_ref, w3_ref, b3_ref,
               e_ref, epad_ref, ecomb_ref):
    z = g_ref[...] + jnp.dot(ef_ref[...], wa_ref[...], preferred_element_type=F32)
    h1 = jnp.maximum(z, 0.0)
    h2 = jnp.dot(h1, w2_ref[...], preferred_element_type=F32) + b2_ref[...]
    en = jnp.dot(jnp.maximum(h2, 0.0), w3_ref[...], preferred_element_type=F32) + b3_ref[...]
    e_ref[...] = en
    epad_ref[...] = jnp.concatenate([en, jnp.zeros_like(en)], axis=-1)

    @pl.when(pl.program_id(0) == 0)
    def _init():
        ecomb_ref[...] = jnp.zeros_like(ecomb_ref)

    ecomb_ref[...] += jnp.sum(en, axis=0, keepdims=True)


def _tc_edge(g, ef, wa, w2, b2, w3, b3):
    nblk = E // EDGE_BLK
    return pl.pallas_call(
        _edge_body,
        grid=(nblk,),
        in_specs=[
            pl.BlockSpec((EDGE_BLK, H1), lambda i: (i, 0)),
            pl.BlockSpec((EDGE_BLK, D_EDGE), lambda i: (i, 0)),
            pl.BlockSpec((D_EDGE, H1), lambda i: (0, 0)),
            pl.BlockSpec((H1, H2), lambda i: (0, 0)),
            pl.BlockSpec((1, H2), lambda i: (0, 0)),
            pl.BlockSpec((H2, D_OUT), lambda i: (0, 0)),
            pl.BlockSpec((1, D_OUT), lambda i: (0, 0)),
        ],
        out_specs=[
            pl.BlockSpec((EDGE_BLK, D_OUT), lambda i: (i, 0)),
            pl.BlockSpec((EDGE_BLK, 2 * D_OUT), lambda i: (i, 0)),
            pl.BlockSpec((1, D_OUT), lambda i: (0, 0)),
        ],
        out_shape=[
            jax.ShapeDtypeStruct((E, D_OUT), F32),
            jax.ShapeDtypeStruct((E, 2 * D_OUT), F32),
            jax.ShapeDtypeStruct((1, D_OUT), F32),
        ],
        compiler_params=pltpu.CompilerParams(
            dimension_semantics=("arbitrary",)),
    )(g, ef, wa, w2, b2, w3, b3)


CS = 40
NCHS = E // CS


def _scatter_body(e_hbm, dst_hbm, out_hbm, id0, id1, rw0, rw1,
                  si0, si1, sr0, sr1, sa0, sa1, acc_sh):
    cid = lax.axis_index("c")
    sid = lax.axis_index("s")
    wid = sid * 2 + cid
    ID, RW = (id0, id1), (rw0, rw1)
    SI, SR, SA = (si0, si1), (sr0, sr1), (sa0, sa1)
    zrows = NPAD // 16
    PT = NCHS // NW

    def zrow(r, _):
        for j in range(8):
            rw0[r, pl.ds(j * 16, 16)] = jnp.zeros((16,), F32)
        return 0

    lax.fori_loop(0, CS, zrow, 0, unroll=2)

    def zcopy(k, _):
        pltpu.sync_copy(rw0, acc_sh.at[pl.ds(sid * zrows + k * CS, CS)])
        return 0

    lax.fori_loop(0, zrows // CS, zcopy, 0)
    plsc.subcore_barrier()

    def ebase(c):
        return (wid + c * NW) * CS

    def issue_loads(c, s):
        pltpu.make_async_copy(dst_hbm.at[pl.ds(ebase(c), CS)], ID[s], SI[s]).start()
        pltpu.make_async_copy(e_hbm.at[pl.ds(ebase(c), CS)], RW[s], SR[s]).start()

    def wait_loads(c, s):
        pltpu.make_async_copy(dst_hbm.at[pl.ds(ebase(c), CS)], ID[s], SI[s]).wait()
        pltpu.make_async_copy(e_hbm.at[pl.ds(ebase(c), CS)], RW[s], SR[s]).wait()

    def issue_scatter(s):
        pltpu.make_async_copy(RW[s], acc_sh.at[ID[s]], SA[s]).start(add=True)

    def wait_scatter(s):
        pltpu.make_async_copy(RW[s], acc_sh.at[ID[s]], SA[s]).wait()

    issue_loads(0, 0)
    issue_loads(1, 1)

    def body(t, _):
        a = 2 * t
        b = a + 1
        wait_loads(a, 0)
        issue_scatter(0)
        wait_loads(b, 1)
        issue_scatter(1)
        wait_scatter(0)

        @pl.when(a + 2 < PT)
        def _la():
            issue_loads(a + 2, 0)

        wait_scatter(1)

        @pl.when(b + 2 < PT)
        def _lb():
            issue_loads(b + 2, 1)

        return 0

    lax.fori_loop(0, PT // 2, body, 0)
    plsc.subcore_barrier()

    @pl.when(sid < 15)
    def _full():
        pltpu.sync_copy(acc_sh.at[pl.ds(sid * zrows, zrows)],
                        out_hbm.at[cid, pl.ds(sid * zrows, zrows)])

    @pl.when(sid == 15)
    def _tail():
        pltpu.sync_copy(acc_sh.at[pl.ds(15 * zrows, N - 15 * zrows)],
                        out_hbm.at[cid, pl.ds(15 * zrows, N - 15 * zrows)])


def _sc_scatter(e_pad, dst):
    mesh = plsc.VectorSubcoreMesh(core_axis_name="c", subcore_axis_name="s")
    fn = functools.partial(
        pl.kernel,
        mesh=mesh,
        out_type=jax.ShapeDtypeStruct((2, N, 2 * D_OUT), F32),
        scratch_types=[
            pltpu.VMEM((CS,), jnp.int32),
            pltpu.VMEM((CS,), jnp.int32),
            pltpu.VMEM((CS, 2 * D_OUT), F32),
            pltpu.VMEM((CS, 2 * D_OUT), F32),
            pltpu.SemaphoreType.DMA,
            pltpu.SemaphoreType.DMA,
            pltpu.SemaphoreType.DMA,
            pltpu.SemaphoreType.DMA,
            pltpu.SemaphoreType.DMA,
            pltpu.SemaphoreType.DMA,
            pltpu.VMEM_SHARED((NPAD, 2 * D_OUT), F32),
        ],
    )(_scatter_body)
    return fn(e_pad, dst)


def _node_body(nf_ref, m0_ref, m1_ref, w1a_ref, w1b_ref, cn_ref,
               w2_ref, b2_ref, w3_ref, b3_ref,
               ecomb_ref, g_ref, wu1_ref, bu1_ref, wu2_ref, bu2_ref,
               wu3_ref, bu3_ref,
               n_ref, uout_ref, ncomb_acc):
    msgs = (m0_ref[...] + m1_ref[...])[:, :D_OUT]
    z = (jnp.dot(nf_ref[...], w1a_ref[...], preferred_element_type=F32)
         + jnp.dot(msgs, w1b_ref[...], preferred_element_type=F32)
         + cn_ref[...])
    h1 = jnp.maximum(z, 0.0)
    h2 = jnp.dot(h1, w2_ref[...], preferred_element_type=F32) + b2_ref[...]
    nn = jnp.dot(jnp.maximum(h2, 0.0), w3_ref[...], preferred_element_type=F32) + b3_ref[...]
    n_ref[...] = nn

    @pl.when(pl.program_id(0) == 0)
    def _init():
        ncomb_acc[...] = jnp.zeros_like(ncomb_acc)

    ncomb_acc[...] += jnp.sum(nn, axis=0, keepdims=True)

    @pl.when(pl.program_id(0) == pl.num_programs(0) - 1)
    def _global():
        inp_u = jnp.concatenate(
            [ncomb_acc[...], ecomb_ref[...], g_ref[...]], axis=-1)
        hu = jnp.maximum(jnp.dot(inp_u, wu1_ref[...], preferred_element_type=F32)
                         + bu1_ref[...], 0.0)
        hu = jnp.dot(hu, wu2_ref[...], preferred_element_type=F32) + bu2_ref[...]
        uout_ref[...] = (jnp.dot(jnp.maximum(hu, 0.0), wu3_ref[...],
                                 preferred_element_type=F32) + bu3_ref[...])


def _tc_node(nf, m0, m1, w1a, w1b, cn, w2, b2, w3, b3,
             ecomb, g, wu1, bu1, wu2, bu2, wu3, bu3):
    nblk = N // NODE_BLK
    full = lambda i: (0, 0)
    return pl.pallas_call(
        _node_body,
        grid=(nblk,),
        in_specs=[
            pl.BlockSpec((NODE_BLK, D_NODE), lambda i: (i, 0)),
            pl.BlockSpec((NODE_BLK, 2 * D_OUT), lambda i: (i, 0)),
            pl.BlockSpec((NODE_BLK, 2 * D_OUT), lambda i: (i, 0)),
            pl.BlockSpec((D_NODE, H1), full),
            pl.BlockSpec((D_OUT, H1), full),
            pl.BlockSpec((1, H1), full),
            pl.BlockSpec((H1, H2), full),
            pl.BlockSpec((1, H2), full),
            pl.BlockSpec((H2, D_OUT), full),
            pl.BlockSpec((1, D_OUT), full),
            pl.BlockSpec((1, D_OUT), full),
            pl.BlockSpec((1, D_U), full),
            pl.BlockSpec((2 * D_OUT + D_U, H1), full),
            pl.BlockSpec((1, H1), full),
            pl.BlockSpec((H1, H2), full),
            pl.BlockSpec((1, H2), full),
            pl.BlockSpec((H2, D_OUT), full),
            pl.BlockSpec((1, D_OUT), full),
        ],
        out_specs=[
            pl.BlockSpec((NODE_BLK, D_OUT), lambda i: (i, 0)),
            pl.BlockSpec((1, D_OUT), full),
        ],
        out_shape=[
            jax.ShapeDtypeStruct((N, D_OUT), F32),
            jax.ShapeDtypeStruct((1, D_OUT), F32),
        ],
        scratch_shapes=[pltpu.VMEM((1, D_OUT), F32)],
        compiler_params=pltpu.CompilerParams(
            dimension_semantics=("arbitrary",)),
    )(nf, m0, m1, w1a, w1b, cn, w2, b2, w3, b3,
      ecomb, g, wu1, bu1, wu2, bu2, wu3, bu3)


def kernel(edge_index, edge_feat, node_feat, g_repr,
           We1, be1, We2, be2, We3, be3,
           Wn1, bn1, Wn2, bn2, Wn3, bn3,
           Wu1, bu1, Wu2, bu2, Wu3, bu3):
    src = edge_index[0].astype(jnp.int32)
    dst = edge_index[1].astype(jnp.int32)

    p, q, cn = _precompute(
        node_feat,
        We1[D_EDGE:D_EDGE + D_NODE],
        We1[D_EDGE + D_NODE:D_EDGE + 2 * D_NODE],
        We1[D_EDGE + 2 * D_NODE:],
        be1.reshape(1, -1),
        Wn1[D_NODE + D_OUT:],
        bn1.reshape(1, -1),
        g_repr,
    )

    g = _sc_gather(p, q, src, dst)

    e_new, e_pad, e_comb = _tc_edge(g, edge_feat, We1[:D_EDGE],
                                    We2, be2.reshape(1, -1),
                                    We3, be3.reshape(1, -1))

    msgs2 = _sc_scatter(e_pad, dst)

    n_new, u_out = _tc_node(
        node_feat, msgs2[0], msgs2[1],
        Wn1[:D_NODE], Wn1[D_NODE:D_NODE + D_OUT], cn,
        Wn2, bn2.reshape(1, -1), Wn3, bn3.reshape(1, -1),
        e_comb, g_repr,
        Wu1, bu1.reshape(1, -1), Wu2, bu2.reshape(1, -1),
        Wu3, bu3.reshape(1, -1),
    )
    return (e_new, n_new, u_out)


# speedup vs baseline: 2.6738x; 1.0595x over previous
import functools

import jax
import jax.numpy as jnp
from jax import lax
from jax.experimental import pallas as pl
from jax.experimental.pallas import tpu as pltpu
from jax.experimental.pallas import tpu_sc as plsc

F32 = jnp.float32

N = 10000
E = 320000
D_NODE = 128
D_EDGE = 16
D_U = 16
H1 = 128
H2 = 64
D_OUT = 64

NW = 32
CG = 40
NCHG = E // CG
NPAD = 10240

EDGE_BLK = 4000
NODE_BLK = 2000
NEBLK = E // EDGE_BLK


def _pq_body(nf_ref, wp_ref, wq_ref, weu_ref, be1_ref, wnu_ref, bn1_ref,
             g_ref, p_ref, q_ref, cn_ref):
    nf = nf_ref[...]
    ce = jnp.dot(g_ref[...], weu_ref[...], preferred_element_type=F32) + be1_ref[...]
    p_ref[...] = jnp.dot(nf, wp_ref[...], preferred_element_type=F32) + ce
    q_ref[...] = jnp.dot(nf, wq_ref[...], preferred_element_type=F32)
    cn_ref[...] = jnp.dot(g_ref[...], wnu_ref[...], preferred_element_type=F32) + bn1_ref[...]


def _precompute(nf, wp, wq, weu, be1, wnu, bn1, g):
    return pl.pallas_call(
        _pq_body,
        out_shape=[
            jax.ShapeDtypeStruct((N, H1), F32),
            jax.ShapeDtypeStruct((N, H1), F32),
            jax.ShapeDtypeStruct((1, H1), F32),
        ],
    )(nf, wp, wq, weu, be1, wnu, bn1, g)


def _gather_body(p_hbm, q_hbm, src_hbm, dst_hbm, out_hbm,
                 is0, is1, id0, id1, rp0, rq0, rp1, rq1,
                 si0, si1, sp0, sp1, sq0, sq1, sw0, sw1):
    cid = lax.axis_index("c")
    sid = lax.axis_index("s")
    wid = sid * 2 + cid
    IS, ID = (is0, is1), (id0, id1)
    RP, RQ = (rp0, rp1), (rq0, rq1)
    SI, SP, SQ, SW = (si0, si1), (sp0, sp1), (sq0, sq1), (sw0, sw1)
    PT = NCHG // NW

    def ebase(c):
        return (wid + c * NW) * CG

    def issue_idx(c, s):
        pltpu.make_async_copy(src_hbm.at[pl.ds(ebase(c), CG)], IS[s], SI[s]).start()
        pltpu.make_async_copy(dst_hbm.at[pl.ds(ebase(c), CG)], ID[s], SI[s]).start()

    def wait_idx(c, s):
        pltpu.make_async_copy(src_hbm.at[pl.ds(ebase(c), CG)], IS[s], SI[s]).wait()
        pltpu.make_async_copy(dst_hbm.at[pl.ds(ebase(c), CG)], ID[s], SI[s]).wait()

    def issue_gather(s):
        pltpu.make_async_copy(p_hbm.at[IS[s]], RP[s], SP[s]).start()
        pltpu.make_async_copy(q_hbm.at[ID[s]], RQ[s], SQ[s]).start()

    def wait_gather(s):
        pltpu.make_async_copy(p_hbm.at[IS[s]], RP[s], SP[s]).wait()
        pltpu.make_async_copy(q_hbm.at[ID[s]], RQ[s], SQ[s]).wait()

    def add(s):
        rp, rq = RP[s], RQ[s]

        def add_row(r, _):
            for j in range(H1 // 16):
                sl = pl.ds(j * 16, 16)
                rp[r, sl] = rp[r, sl] + rq[r, sl]
            return 0

        lax.fori_loop(0, CG, add_row, 0, unroll=2)

    def issue_write(c, s):
        pltpu.make_async_copy(RP[s], out_hbm.at[pl.ds(ebase(c), CG)], SW[s]).start()

    def wait_write(c, s):
        pltpu.make_async_copy(RP[s], out_hbm.at[pl.ds(ebase(c), CG)], SW[s]).wait()

    issue_idx(0, 0)
    issue_idx(1, 1)
    wait_idx(0, 0)
    issue_gather(0)

    def body(t, _):
        a = 2 * t
        b = a + 1
        wait_idx(b, 1)

        @pl.when(t > 0)
        def _wb1():
            wait_write(b - 2, 1)

        issue_gather(1)
        wait_gather(0)

        @pl.when(a + 2 < PT)
        def _ia():
            issue_idx(a + 2, 0)

        add(0)
        issue_write(a, 0)
        wait_gather(1)

        @pl.when(b + 2 < PT)
        def _ib():
            issue_idx(b + 2, 1)

        add(1)
        issue_write(b, 1)

        @pl.when(a + 2 < PT)
        def _ga():
            wait_idx(a + 2, 0)
            wait_write(a, 0)
            issue_gather(0)

        return 0

    lax.fori_loop(0, PT // 2, body, 0)
    wait_write(PT - 2, 0)
    wait_write(PT - 1, 1)


def _sc_gather(p, q, src, dst):
    mesh = plsc.VectorSubcoreMesh(core_axis_name="c", subcore_axis_name="s")
    fn = functools.partial(
        pl.kernel,
        mesh=mesh,
        out_type=jax.ShapeDtypeStruct((E, H1), F32),
        scratch_types=[
            pltpu.VMEM((CG,), jnp.int32),
            pltpu.VMEM((CG,), jnp.int32),
            pltpu.VMEM((CG,), jnp.int32),
            pltpu.VMEM((CG,), jnp.int32),
            pltpu.VMEM((CG, H1), F32),
            pltpu.VMEM((CG, H1), F32),
            pltpu.VMEM((CG, H1), F32),
            pltpu.VMEM((CG, H1), F32),
            pltpu.SemaphoreType.DMA,
            pltpu.SemaphoreType.DMA,
            pltpu.SemaphoreType.DMA,
            pltpu.SemaphoreType.DMA,
            pltpu.SemaphoreType.DMA,
            pltpu.SemaphoreType.DMA,
            pltpu.SemaphoreType.DMA,
            pltpu.SemaphoreType.DMA,
        ],
    )(_gather_body)
    return fn(p, q, src, dst)


def _edge_body(g_ref, ef_ref, wa_ref, w2_ref, b2_ref, w3_ref, b3_ref,
               e_ref, epad_ref, ecomb_ref):
    z = g_ref[...] + jnp.dot(ef_ref[...], wa_ref[...], preferred_element_type=F32)
    h1 = jnp.maximum(z, 0.0)
    h2 = jnp.dot(h1, w2_ref[...], preferred_element_type=F32) + b2_ref[...]
    en = jnp.dot(jnp.maximum(h2, 0.0), w3_ref[...], preferred_element_type=F32) + b3_ref[...]
    e_ref[...] = en
    epad_ref[...] = jnp.concatenate([en, jnp.zeros_like(en)], axis=-1)
    ecomb_ref[...] = jnp.sum(en, axis=0, keepdims=True)[None]


def _tc_edge(g, ef, wa, w2, b2, w3, b3):
    nblk = E // EDGE_BLK
    return pl.pallas_call(
        _edge_body,
        grid=(nblk,),
        in_specs=[
            pl.BlockSpec((EDGE_BLK, H1), lambda i: (i, 0)),
            pl.BlockSpec((EDGE_BLK, D_EDGE), lambda i: (i, 0)),
            pl.BlockSpec((D_EDGE, H1), lambda i: (0, 0)),
            pl.BlockSpec((H1, H2), lambda i: (0, 0)),
            pl.BlockSpec((1, H2), lambda i: (0, 0)),
            pl.BlockSpec((H2, D_OUT), lambda i: (0, 0)),
            pl.BlockSpec((1, D_OUT), lambda i: (0, 0)),
        ],
        out_specs=[
            pl.BlockSpec((EDGE_BLK, D_OUT), lambda i: (i, 0)),
            pl.BlockSpec((EDGE_BLK, 2 * D_OUT), lambda i: (i, 0)),
            pl.BlockSpec((1, 1, D_OUT), lambda i: (i, 0, 0)),
        ],
        out_shape=[
            jax.ShapeDtypeStruct((E, D_OUT), F32),
            jax.ShapeDtypeStruct((E, 2 * D_OUT), F32),
            jax.ShapeDtypeStruct((nblk, 1, D_OUT), F32),
        ],
        compiler_params=pltpu.CompilerParams(
            dimension_semantics=("arbitrary",)),
    )(g, ef, wa, w2, b2, w3, b3)


CS = 40
NCHS = E // CS


def _scatter_body(e_hbm, dst_hbm, out_hbm, id0, id1, rw0, rw1,
                  si0, si1, sr0, sr1, sa0, sa1, acc_sh):
    cid = lax.axis_index("c")
    sid = lax.axis_index("s")
    wid = sid * 2 + cid
    ID, RW = (id0, id1), (rw0, rw1)
    SI, SR, SA = (si0, si1), (sr0, sr1), (sa0, sa1)
    zrows = NPAD // 16
    PT = NCHS // NW

    def zrow(r, _):
        for j in range(8):
            rw0[r, pl.ds(j * 16, 16)] = jnp.zeros((16,), F32)
        return 0

    lax.fori_loop(0, CS, zrow, 0, unroll=2)

    def zcopy(k, _):
        pltpu.sync_copy(rw0, acc_sh.at[pl.ds(sid * zrows + k * CS, CS)])
        return 0

    lax.fori_loop(0, zrows // CS, zcopy, 0)
    plsc.subcore_barrier()

    def ebase(c):
        return (wid + c * NW) * CS

    def issue_loads(c, s):
        pltpu.make_async_copy(dst_hbm.at[pl.ds(ebase(c), CS)], ID[s], SI[s]).start()
        pltpu.make_async_copy(e_hbm.at[pl.ds(ebase(c), CS)], RW[s], SR[s]).start()

    def wait_loads(c, s):
        pltpu.make_async_copy(dst_hbm.at[pl.ds(ebase(c), CS)], ID[s], SI[s]).wait()
        pltpu.make_async_copy(e_hbm.at[pl.ds(ebase(c), CS)], RW[s], SR[s]).wait()

    def issue_scatter(s):
        pltpu.make_async_copy(RW[s], acc_sh.at[ID[s]], SA[s]).start(add=True)

    def wait_scatter(s):
        pltpu.make_async_copy(RW[s], acc_sh.at[ID[s]], SA[s]).wait()

    issue_loads(0, 0)
    issue_loads(1, 1)

    def body(t, _):
        a = 2 * t
        b = a + 1
        wait_loads(a, 0)
        issue_scatter(0)
        wait_loads(b, 1)
        issue_scatter(1)
        wait_scatter(0)

        @pl.when(a + 2 < PT)
        def _la():
            issue_loads(a + 2, 0)

        wait_scatter(1)

        @pl.when(b + 2 < PT)
        def _lb():
            issue_loads(b + 2, 1)

        return 0

    lax.fori_loop(0, PT // 2, body, 0)
    plsc.subcore_barrier()

    @pl.when(sid < 15)
    def _full():
        pltpu.sync_copy(acc_sh.at[pl.ds(sid * zrows, zrows)],
                        out_hbm.at[cid, pl.ds(sid * zrows, zrows)])

    @pl.when(sid == 15)
    def _tail():
        pltpu.sync_copy(acc_sh.at[pl.ds(15 * zrows, N - 15 * zrows)],
                        out_hbm.at[cid, pl.ds(15 * zrows, N - 15 * zrows)])


def _sc_scatter(e_pad, dst):
    mesh = plsc.VectorSubcoreMesh(core_axis_name="c", subcore_axis_name="s")
    fn = functools.partial(
        pl.kernel,
        mesh=mesh,
        out_type=jax.ShapeDtypeStruct((2, N, 2 * D_OUT), F32),
        scratch_types=[
            pltpu.VMEM((CS,), jnp.int32),
            pltpu.VMEM((CS,), jnp.int32),
            pltpu.VMEM((CS, 2 * D_OUT), F32),
            pltpu.VMEM((CS, 2 * D_OUT), F32),
            pltpu.SemaphoreType.DMA,
            pltpu.SemaphoreType.DMA,
            pltpu.SemaphoreType.DMA,
            pltpu.SemaphoreType.DMA,
            pltpu.SemaphoreType.DMA,
            pltpu.SemaphoreType.DMA,
            pltpu.VMEM_SHARED((NPAD, 2 * D_OUT), F32),
        ],
    )(_scatter_body)
    return fn(e_pad, dst)


def _node_body(nf_ref, m0_ref, m1_ref, w1a_ref, w1b_ref, cn_ref,
               w2_ref, b2_ref, w3_ref, b3_ref,
               ecomb_ref, g_ref, wu1_ref, bu1_ref, wu2_ref, bu2_ref,
               wu3_ref, bu3_ref,
               n_ref, uout_ref, ncomb_acc):
    msgs = (m0_ref[...] + m1_ref[...])[:, :D_OUT]
    z = (jnp.dot(nf_ref[...], w1a_ref[...], preferred_element_type=F32)
         + jnp.dot(msgs, w1b_ref[...], preferred_element_type=F32)
         + cn_ref[...])
    h1 = jnp.maximum(z, 0.0)
    h2 = jnp.dot(h1, w2_ref[...], preferred_element_type=F32) + b2_ref[...]
    nn = jnp.dot(jnp.maximum(h2, 0.0), w3_ref[...], preferred_element_type=F32) + b3_ref[...]
    n_ref[...] = nn

    @pl.when(pl.program_id(0) == 0)
    def _init():
        ncomb_acc[...] = jnp.zeros_like(ncomb_acc)

    ncomb_acc[...] += jnp.sum(nn, axis=0, keepdims=True)

    @pl.when(pl.program_id(0) == pl.num_programs(0) - 1)
    def _global():
        ecomb = jnp.sum(ecomb_ref[...], axis=0)
        inp_u = jnp.concatenate(
            [ncomb_acc[...], ecomb, g_ref[...]], axis=-1)
        hu = jnp.maximum(jnp.dot(inp_u, wu1_ref[...], preferred_element_type=F32)
                         + bu1_ref[...], 0.0)
        hu = jnp.dot(hu, wu2_ref[...], preferred_element_type=F32) + bu2_ref[...]
        uout_ref[...] = (jnp.dot(jnp.maximum(hu, 0.0), wu3_ref[...],
                                 preferred_element_type=F32) + bu3_ref[...])


def _tc_node(nf, m0, m1, w1a, w1b, cn, w2, b2, w3, b3,
             ecomb, g, wu1, bu1, wu2, bu2, wu3, bu3):
    nblk = N // NODE_BLK
    full = lambda i: (0, 0)
    return pl.pallas_call(
        _node_body,
        grid=(nblk,),
        in_specs=[
            pl.BlockSpec((NODE_BLK, D_NODE), lambda i: (i, 0)),
            pl.BlockSpec((NODE_BLK, 2 * D_OUT), lambda i: (i, 0)),
            pl.BlockSpec((NODE_BLK, 2 * D_OUT), lambda i: (i, 0)),
            pl.BlockSpec((D_NODE, H1), full),
            pl.BlockSpec((D_OUT, H1), full),
            pl.BlockSpec((1, H1), full),
            pl.BlockSpec((H1, H2), full),
            pl.BlockSpec((1, H2), full),
            pl.BlockSpec((H2, D_OUT), full),
            pl.BlockSpec((1, D_OUT), full),
            pl.BlockSpec((NEBLK, 1, D_OUT), lambda i: (0, 0, 0)),
            pl.BlockSpec((1, D_U), full),
            pl.BlockSpec((2 * D_OUT + D_U, H1), full),
            pl.BlockSpec((1, H1), full),
            pl.BlockSpec((H1, H2), full),
            pl.BlockSpec((1, H2), full),
            pl.BlockSpec((H2, D_OUT), full),
            pl.BlockSpec((1, D_OUT), full),
        ],
        out_specs=[
            pl.BlockSpec((NODE_BLK, D_OUT), lambda i: (i, 0)),
            pl.BlockSpec((1, D_OUT), full),
        ],
        out_shape=[
            jax.ShapeDtypeStruct((N, D_OUT), F32),
            jax.ShapeDtypeStruct((1, D_OUT), F32),
        ],
        scratch_shapes=[pltpu.VMEM((1, D_OUT), F32)],
        compiler_params=pltpu.CompilerParams(
            dimension_semantics=("arbitrary",)),
    )(nf, m0, m1, w1a, w1b, cn, w2, b2, w3, b3,
      ecomb, g, wu1, bu1, wu2, bu2, wu3, bu3)


def kernel(edge_index, edge_feat, node_feat, g_repr,
           We1, be1, We2, be2, We3, be3,
           Wn1, bn1, Wn2, bn2, Wn3, bn3,
           Wu1, bu1, Wu2, bu2, Wu3, bu3):
    src = edge_index[0].astype(jnp.int32)
    dst = edge_index[1].astype(jnp.int32)

    p, q, cn = _precompute(
        node_feat,
        We1[D_EDGE:D_EDGE + D_NODE],
        We1[D_EDGE + D_NODE:D_EDGE + 2 * D_NODE],
        We1[D_EDGE + 2 * D_NODE:],
        be1.reshape(1, -1),
        Wn1[D_NODE + D_OUT:],
        bn1.reshape(1, -1),
        g_repr,
    )

    g = _sc_gather(p, q, src, dst)

    e_new, e_pad, e_comb = _tc_edge(g, edge_feat, We1[:D_EDGE],
                                    We2, be2.reshape(1, -1),
                                    We3, be3.reshape(1, -1))

    msgs2 = _sc_scatter(e_pad, dst)

    n_new, u_out = _tc_node(
        node_feat, msgs2[0], msgs2[1],
        Wn1[:D_NODE], Wn1[D_NODE:D_NODE + D_OUT], cn,
        Wn2, bn2.reshape(1, -1), Wn3, bn3.reshape(1, -1),
        e_comb, g_repr,
        Wu1, bu1.reshape(1, -1), Wu2, bu2.reshape(1, -1),
        Wu3, bu3.reshape(1, -1),
    )
    return (e_new, n_new, u_out)
